# baseline (speedup 1.0000x reference)
import functools

import jax
import jax.numpy as jnp
from jax import lax
from jax.experimental import pallas as pl
from jax.experimental.pallas import tpu as pltpu
from jax.experimental.pallas import tpu_sc as plsc

D = 32
EPS = 1e-5

NC = 2
NS = 16
NW = NC * NS

IDX_MINOR = 128
GATHERS_PER_CHUNK = 8
CHUNK = IDX_MINOR * GATHERS_PER_CHUNK


def _rsqrt_sc(v):
    i = plsc.bitcast(v, jnp.int32)
    i = jnp.int32(0x5F3759DF) - lax.shift_right_arithmetic(i, 1)
    y = plsc.bitcast(i, jnp.float32)
    y = y * (1.5 - 0.5 * v * y * y)
    y = y * (1.5 - 0.5 * v * y * y)
    y = y * (1.5 - 0.5 * v * y * y)
    return y


def _sc_fused(xp, table, gb, bb, L, B):
    n_idx_rows = xp.shape[1]
    n_per_w = n_idx_rows * IDX_MINOR
    n_chunks = n_per_w // CHUNK
    kb_per_l = B // CHUNK
    n_groups = CHUNK // 16

    mesh = plsc.VectorSubcoreMesh(core_axis_name="c", subcore_axis_name="s")

    @functools.partial(
        pl.kernel,
        mesh=mesh,
        out_type=jax.ShapeDtypeStruct((L, 4, kb_per_l, 8192), jnp.float32),
        compiler_params=pltpu.CompilerParams(
            use_tc_tiling_on_sc=False, needs_layout_passes=False
        ),
        scratch_types=[
            pltpu.VMEM((n_idx_rows, IDX_MINOR), jnp.int32),
            pltpu.VMEM((CHUNK, D), jnp.float32),
            pltpu.VMEM((4, 8192), jnp.float32),
            pltpu.VMEM((D, 16), jnp.float32),
            pltpu.VMEM((D, 16), jnp.float32),
            pltpu.SemaphoreType.DMA,
        ],
    )
    def k(x_hbm, table_hbm, gb_hbm, bb_hbm, out_hbm, idx_v, rows_v, tbuf, gb_v, bb_v, sem):
        iota16 = lax.iota(jnp.int32, 16)
        wid = lax.axis_index("c") * NS + lax.axis_index("s")
        ch0 = wid * n_chunks
        pltpu.sync_copy(x_hbm.at[wid], idx_v)
        pltpu.sync_copy(gb_hbm, gb_v)
        pltpu.sync_copy(bb_hbm, bb_v)

        def chunk_body(c, carry):
            ch = ch0 + c
            l = ch // kb_per_l
            kb = ch % kb_per_l
            descs = []
            for j in range(GATHERS_PER_CHUNK):
                descs.append(
                    pltpu.async_copy(
                        table_hbm.at[idx_v.at[c * GATHERS_PER_CHUNK + j]],
                        rows_v.at[pl.ds(j * IDX_MINOR, IDX_MINOR)],
                        sem,
                    )
                )
            for dsc in descs:
                dsc.wait()

            def group_body(grp, carry2):
                t0 = grp * 16
                toks = t0 + iota16
                vs = []
                s = jnp.zeros((16,), jnp.float32)
                s2 = jnp.zeros((16,), jnp.float32)
                for d in range(D):
                    v = plsc.load_gather(
                        rows_v, [toks, jnp.full((16,), d, jnp.int32)]
                    )
                    vs.append(v)
                    s = s + v
                    s2 = s2 + v * v
                mean = s * (1.0 / D)
                var = s2 * (1.0 / D) - mean * mean
                a = _rsqrt_sc(var + EPS)
                ma = mean * a
                off0 = (grp // 8) * 1024 + (grp % 8) * 16
                for d in range(D):
                    y = vs[d] * a - ma
                    y = y * gb_v[d] + bb_v[d]
                    y = jnp.maximum(y, 0.0)
                    tbuf[d // 8, pl.ds(off0 + (d % 8) * 128, 16)] = y
                return carry2

            lax.fori_loop(0, n_groups, group_body, 0)
            for dt in range(4):
                pltpu.sync_copy(tbuf.at[dt], out_hbm.at[l, dt, kb])
            return carry

        lax.fori_loop(0, n_chunks, chunk_body, 0)

    return k(xp, table, gb, bb)


def kernel(x, table, gamma, beta):
    B, L = x.shape
    N = B * L
    xp = x.T.reshape(NW, N // (NW * IDX_MINOR), IDX_MINOR)
    gb = jnp.broadcast_to(gamma[:, None], (D, 16))
    bb = jnp.broadcast_to(beta[:, None], (D, 16))
    g = _sc_fused(xp, table, gb, bb, L, B)
    g6 = g.reshape(L, 4, B // 128, 8, 128)
    out3 = g6.transpose(0, 1, 3, 2, 4).reshape(L, D, B)
    return jnp.transpose(out3, (2, 0, 1))


# speedup vs baseline: 1.0044x; 1.0044x over previous
import functools

import jax
import jax.numpy as jnp
from jax import lax
from jax.experimental import pallas as pl
from jax.experimental.pallas import tpu as pltpu
from jax.experimental.pallas import tpu_sc as plsc

D = 32
EPS = 1e-5

NC = 2
NS = 16
NW = NC * NS

IDX_MINOR = 128
GATHERS_PER_CHUNK = 8
CHUNK = IDX_MINOR * GATHERS_PER_CHUNK


def _rsqrt_sc(v):
    i = plsc.bitcast(v, jnp.int32)
    i = jnp.int32(0x5F3759DF) - lax.shift_right_arithmetic(i, 1)
    y = plsc.bitcast(i, jnp.float32)
    y = y * (1.5 - 0.5 * v * y * y)
    y = y * (1.5 - 0.5 * v * y * y)
    y = y * (1.5 - 0.5 * v * y * y)
    return y


def _sc_fused(xp, table, gb, bb, L, B):
    n_idx_rows = xp.shape[1]
    n_per_w = n_idx_rows * IDX_MINOR
    n_chunks = n_per_w // CHUNK
    kb_per_l = B // CHUNK
    n_groups = CHUNK // 16

    mesh = plsc.VectorSubcoreMesh(core_axis_name="c", subcore_axis_name="s")

    @functools.partial(
        pl.kernel,
        mesh=mesh,
        out_type=jax.ShapeDtypeStruct((L, 4, kb_per_l, 8192), jnp.float32),
        compiler_params=pltpu.CompilerParams(
            use_tc_tiling_on_sc=False, needs_layout_passes=False
        ),
        scratch_types=[
            pltpu.VMEM((n_idx_rows, IDX_MINOR), jnp.int32),
            pltpu.VMEM((CHUNK, D), jnp.float32),
            pltpu.VMEM((4, 8192), jnp.float32),
            pltpu.VMEM((D, 16), jnp.float32),
            pltpu.VMEM((D, 16), jnp.float32),
            pltpu.SemaphoreType.DMA,
        ],
    )
    def k(x_hbm, table_hbm, gb_hbm, bb_hbm, out_hbm, idx_v, rows_v, tbuf, gb_v, bb_v, sem):
        iota16 = lax.iota(jnp.int32, 16)
        wid = lax.axis_index("c") * NS + lax.axis_index("s")
        ch0 = wid * n_chunks
        pltpu.sync_copy(x_hbm.at[wid], idx_v)
        pltpu.sync_copy(gb_hbm, gb_v)
        pltpu.sync_copy(bb_hbm, bb_v)

        def chunk_body(c, carry):
            ch = ch0 + c
            l = ch // kb_per_l
            kb = ch % kb_per_l
            descs = []
            for j in range(GATHERS_PER_CHUNK):
                descs.append(
                    pltpu.async_copy(
                        table_hbm.at[idx_v.at[c * GATHERS_PER_CHUNK + j]],
                        rows_v.at[pl.ds(j * IDX_MINOR, IDX_MINOR)],
                        sem,
                    )
                )
            for dsc in descs:
                dsc.wait()

            @plsc.parallel_loop(0, n_groups, 1, unroll=2)
            def group_body(grp):
                t0 = grp * 16
                toks = t0 + iota16
                vs = [
                    plsc.load_gather(
                        rows_v, [toks, jnp.full((16,), d, jnp.int32)]
                    )
                    for d in range(D)
                ]
                sq = [v * v for v in vs]
                ps, ps2 = list(vs), sq
                while len(ps) > 1:
                    ps = [ps[i] + ps[i + 1] for i in range(0, len(ps), 2)]
                    ps2 = [ps2[i] + ps2[i + 1] for i in range(0, len(ps2), 2)]
                mean = ps[0] * (1.0 / D)
                var = ps2[0] * (1.0 / D) - mean * mean
                a = _rsqrt_sc(var + EPS)
                ma = mean * a
                off0 = (grp // 8) * 1024 + (grp % 8) * 16
                for d in range(D):
                    y = vs[d] * a - ma
                    y = y * gb_v[d] + bb_v[d]
                    y = jnp.maximum(y, 0.0)
                    tbuf[d // 8, pl.ds(off0 + (d % 8) * 128, 16)] = y
            for dt in range(4):
                pltpu.sync_copy(tbuf.at[dt], out_hbm.at[l, dt, kb])
            return carry

        lax.fori_loop(0, n_chunks, chunk_body, 0)

    return k(xp, table, gb, bb)


def kernel(x, table, gamma, beta):
    B, L = x.shape
    N = B * L
    xp = x.T.reshape(NW, N // (NW * IDX_MINOR), IDX_MINOR)
    gb = jnp.broadcast_to(gamma[:, None], (D, 16))
    bb = jnp.broadcast_to(beta[:, None], (D, 16))
    g = _sc_fused(xp, table, gb, bb, L, B)
    g6 = g.reshape(L, 4, B // 128, 8, 128)
    out3 = g6.transpose(0, 1, 3, 2, 4).reshape(L, D, B)
    return jnp.transpose(out3, (2, 0, 1))


# speedup vs baseline: 1.1115x; 1.1066x over previous
import functools

import jax
import jax.numpy as jnp
from jax import lax
from jax.experimental import pallas as pl
from jax.experimental.pallas import tpu as pltpu
from jax.experimental.pallas import tpu_sc as plsc

D = 32
EPS = 1e-5

NC = 2
NS = 16
NW = NC * NS

IDX_MINOR = 128
GATHERS_PER_CHUNK = 8
CHUNK = IDX_MINOR * GATHERS_PER_CHUNK


def _rsqrt_sc(v):
    i = plsc.bitcast(v, jnp.int32)
    i = jnp.int32(0x5F3759DF) - lax.shift_right_arithmetic(i, 1)
    y = plsc.bitcast(i, jnp.float32)
    y = y * (1.5 - 0.5 * v * y * y)
    y = y * (1.5 - 0.5 * v * y * y)
    y = y * (1.5 - 0.5 * v * y * y)
    return y


def _sc_fused(xp, table, gb, bb, L, B):
    n_idx_rows = xp.shape[1]
    n_per_w = n_idx_rows * IDX_MINOR
    n_chunks = n_per_w // CHUNK
    kb_per_l = B // CHUNK
    n_groups = CHUNK // 16

    mesh = plsc.VectorSubcoreMesh(core_axis_name="c", subcore_axis_name="s")

    @functools.partial(
        pl.kernel,
        mesh=mesh,
        out_type=jax.ShapeDtypeStruct((L, 4, kb_per_l, 8192), jnp.float32),
        compiler_params=pltpu.CompilerParams(
            use_tc_tiling_on_sc=False, needs_layout_passes=False
        ),
        scratch_types=[
            pltpu.VMEM((n_idx_rows, IDX_MINOR), jnp.int32),
            pltpu.VMEM((CHUNK, D), jnp.float32),
            pltpu.VMEM((4, 8192), jnp.float32),
            pltpu.VMEM((D, 16), jnp.float32),
            pltpu.VMEM((D, 16), jnp.float32),
            pltpu.SemaphoreType.DMA,
        ],
    )
    def k(x_hbm, table_hbm, gb_hbm, bb_hbm, out_hbm, idx_v, rows_v, tbuf, gb_v, bb_v, sem):
        iota16 = lax.iota(jnp.int32, 16)
        wid = lax.axis_index("c") * NS + lax.axis_index("s")
        ch0 = wid * n_chunks
        pltpu.sync_copy(x_hbm.at[wid], idx_v)
        pltpu.sync_copy(gb_hbm, gb_v)
        pltpu.sync_copy(bb_hbm, bb_v)

        @pl.loop(0, n_chunks, unroll=1)
        def chunk_body(c):
            ch = ch0 + c
            l = ch // kb_per_l
            kb = ch % kb_per_l
            descs = []
            for j in range(GATHERS_PER_CHUNK):
                descs.append(
                    pltpu.async_copy(
                        table_hbm.at[idx_v.at[c * GATHERS_PER_CHUNK + j]],
                        rows_v.at[pl.ds(j * IDX_MINOR, IDX_MINOR)],
                        sem,
                    )
                )
            for dsc in descs:
                dsc.wait()

            @plsc.parallel_loop(0, n_groups, 1, unroll=1)
            def group_body(grp):
                t0 = grp * 16
                toks = t0 + iota16
                vs = [
                    plsc.load_gather(
                        rows_v, [toks, jnp.full((16,), d, jnp.int32)]
                    )
                    for d in range(D)
                ]
                sq = [v * v for v in vs]
                ps, ps2 = list(vs), sq
                while len(ps) > 1:
                    ps = [ps[i] + ps[i + 1] for i in range(0, len(ps), 2)]
                    ps2 = [ps2[i] + ps2[i + 1] for i in range(0, len(ps2), 2)]
                mean = ps[0] * (1.0 / D)
                var = ps2[0] * (1.0 / D) - mean * mean
                a = _rsqrt_sc(var + EPS)
                ma = mean * a
                off0 = (grp // 8) * 1024 + (grp % 8) * 16
                for d in range(D):
                    y = vs[d] * a - ma
                    y = y * gb_v[d] + bb_v[d]
                    y = jnp.maximum(y, 0.0)
                    tbuf[d // 8, pl.ds(off0 + (d % 8) * 128, 16)] = y
            for dt in range(4):
                pltpu.sync_copy(tbuf.at[dt], out_hbm.at[l, dt, kb])

    return k(xp, table, gb, bb)


def kernel(x, table, gamma, beta):
    B, L = x.shape
    N = B * L
    xp = x.T.reshape(NW, N // (NW * IDX_MINOR), IDX_MINOR)
    gb = jnp.broadcast_to(gamma[:, None], (D, 16))
    bb = jnp.broadcast_to(beta[:, None], (D, 16))
    g = _sc_fused(xp, table, gb, bb, L, B)
    g6 = g.reshape(L, 4, B // 128, 8, 128)
    out3 = g6.transpose(0, 1, 3, 2, 4).reshape(L, D, B)
    return jnp.transpose(out3, (2, 0, 1))


# speedup vs baseline: 1.5001x; 1.3497x over previous
import functools

import jax
import jax.numpy as jnp
import numpy as np
from jax import lax
from jax.experimental import pallas as pl
from jax.experimental.pallas import tpu as pltpu
from jax.experimental.pallas import tpu_sc as plsc

D = 32
EPS = 1e-5

NC = 2
NS = 16
NW = NC * NS

IDX_MINOR = 128
GATHERS_PER_CHUNK = 8
CHUNK = IDX_MINOR * GATHERS_PER_CHUNK

TBLK = 8192
TSUB = TBLK // 4

KB = 2048
RB = KB // 4


def _tc_pack_table(table_t, n_blocks):
    def body(z_ref, o_ref):
        z = z_ref[...]
        for q in range(4):
            o_ref[:, q * D:(q + 1) * D] = z[:, q * TSUB:(q + 1) * TSUB].T

    return pl.pallas_call(
        body,
        grid=(n_blocks,),
        in_specs=[pl.BlockSpec((D, TBLK), lambda j: (0, j))],
        out_specs=pl.BlockSpec((TSUB, 128), lambda j: (j, 0)),
        out_shape=jax.ShapeDtypeStruct((n_blocks * TSUB, 128), jnp.float32),
    )(table_t)


def _sc_gather(x_grouped, table):
    n_idx_rows = x_grouped.shape[1]
    n_per_w = n_idx_rows * IDX_MINOR
    n_chunks = n_per_w // CHUNK
    N = NW * n_per_w

    mesh = plsc.VectorSubcoreMesh(core_axis_name="c", subcore_axis_name="s")

    @functools.partial(
        pl.kernel,
        mesh=mesh,
        out_type=jax.ShapeDtypeStruct((N, D), jnp.float32),
        compiler_params=pltpu.CompilerParams(use_tc_tiling_on_sc=False),
        scratch_types=[
            pltpu.VMEM((n_idx_rows, IDX_MINOR), jnp.int32),
            pltpu.VMEM((CHUNK, D), jnp.float32),
            pltpu.SemaphoreType.DMA,
        ],
    )
    def k(x_hbm, table_hbm, out_hbm, idx_v, rows_v, sem):
        wid = lax.axis_index("c") * NS + lax.axis_index("s")
        base = wid * n_per_w
        pltpu.sync_copy(x_hbm.at[wid], idx_v)

        @pl.loop(0, n_chunks, unroll=1)
        def chunk_body(c):
            descs = []
            for j in range(GATHERS_PER_CHUNK):
                descs.append(
                    pltpu.async_copy(
                        table_hbm.at[idx_v.at[c * GATHERS_PER_CHUNK + j]],
                        rows_v.at[pl.ds(j * IDX_MINOR, IDX_MINOR)],
                        sem,
                    )
                )
            for dsc in descs:
                dsc.wait()
            pltpu.sync_copy(rows_v, out_hbm.at[pl.ds(base + c * CHUNK, CHUNK)])

    return k(x_grouped, table)


def _tc_norm_t(z4, seg, gt, bt, L, B):
    nkb = B // KB

    def body(z_ref, seg_ref, g_ref, b_ref, o_ref):
        z = z_ref[...]
        s = seg_ref[...]
        s1 = jnp.dot(z, s, preferred_element_type=jnp.float32)
        s2 = jnp.dot(z * z, s, preferred_element_type=jnp.float32)
        mean = s1 * (1.0 / D)
        var = s2 * (1.0 / D) - mean * mean
        rstd = lax.rsqrt(var + EPS)
        y = jnp.maximum((z - mean) * rstd * g_ref[...] + b_ref[...], 0.0)
        for sseg in range(4):
            o_ref[0, :, sseg * RB:(sseg + 1) * RB] = y[:, sseg * D:(sseg + 1) * D].T

    return pl.pallas_call(
        body,
        grid=(L, nkb),
        in_specs=[
            pl.BlockSpec((RB, 128), lambda l, kb: (l * nkb + kb, 0)),
            pl.BlockSpec((128, 128), lambda l, kb: (0, 0)),
            pl.BlockSpec((1, 128), lambda l, kb: (0, 0)),
            pl.BlockSpec((1, 128), lambda l, kb: (0, 0)),
        ],
        out_specs=pl.BlockSpec((1, D, KB), lambda l, kb: (l, 0, kb)),
        out_shape=jax.ShapeDtypeStruct((L, D, B), jnp.float32),
    )(z4, seg, gt, bt)


def kernel(x, table, gamma, beta):
    B, L = x.shape
    V = table.shape[0]
    N = B * L
    n_blocks = (V + TBLK - 1) // TBLK
    nkb = B // KB

    t4 = _tc_pack_table(table.T, n_blocks)
    tpack = t4.reshape(n_blocks * TBLK, D)

    xq = (
        jnp.left_shift(jnp.right_shift(x, 13), 13)
        | jnp.left_shift(x & 2047, 2)
        | (jnp.right_shift(x, 11) & 3)
    )
    xp = (
        xq.T.reshape(L, nkb, 4, RB)
        .transpose(0, 1, 3, 2)
        .reshape(NW, N // (NW * IDX_MINOR), IDX_MINOR)
    )
    g = _sc_gather(xp, tpack)

    z4 = g.reshape(N // 4, 4 * D)
    seg = jnp.asarray(
        (np.arange(128)[:, None] // D) == (np.arange(128)[None, :] // D),
        dtype=jnp.float32,
    )
    gt = jnp.tile(gamma, 4).reshape(1, 128)
    bt = jnp.tile(beta, 4).reshape(1, 128)
    out_t = _tc_norm_t(z4, seg, gt, bt, L, B)
    return jnp.transpose(out_t, (2, 0, 1))


# speedup vs baseline: 2.0406x; 1.3603x over previous
import functools

import jax
import jax.numpy as jnp
import numpy as np
from jax import lax
from jax.experimental import pallas as pl
from jax.experimental.pallas import tpu as pltpu
from jax.experimental.pallas import tpu_sc as plsc

D = 32
EPS = 1e-5

NC = 2
NS = 16
NW = NC * NS

IDX_MINOR = 128
GATHERS_PER_CHUNK = 8
CHUNK = IDX_MINOR * GATHERS_PER_CHUNK

TBLK = 8192
TSUB = TBLK // 4

KB = 2048
RB = KB // 4


def _tc_pack_table(table_t, ident, n_blocks):
    def body(z_ref, e_ref, o_ref):
        z = z_ref[...]
        acc = None
        for q in range(4):
            zq = z[:, q * TSUB:(q + 1) * TSUB]
            eq = e_ref[pl.ds(q * D, D), :]
            t = lax.dot_general(
                zq, eq, (((0,), (0,)), ((), ())),
                preferred_element_type=jnp.float32,
            )
            acc = t if acc is None else acc + t
        o_ref[...] = acc

    return pl.pallas_call(
        body,
        grid=(n_blocks,),
        in_specs=[
            pl.BlockSpec((D, TBLK), lambda j: (0, j)),
            pl.BlockSpec((128, 128), lambda j: (0, 0)),
        ],
        out_specs=pl.BlockSpec((TSUB, 128), lambda j: (j, 0)),
        out_shape=jax.ShapeDtypeStruct((n_blocks * TSUB, 128), jnp.float32),
        compiler_params=pltpu.CompilerParams(fuse_transposed_lhs_in_matmul=True),
    )(table_t, ident)


def _sc_gather(x_grouped, table, L, B):
    n_idx_rows = x_grouped.shape[1]
    n_per_w = n_idx_rows * IDX_MINOR
    n_chunks = n_per_w // CHUNK
    kb_per_l = B // KB

    mesh = plsc.VectorSubcoreMesh(core_axis_name="c", subcore_axis_name="s")

    @functools.partial(
        pl.kernel,
        mesh=mesh,
        out_type=jax.ShapeDtypeStruct((L, kb_per_l, RB, 4, D), jnp.float32),
        compiler_params=pltpu.CompilerParams(use_tc_tiling_on_sc=False),
        scratch_types=[
            pltpu.VMEM((n_idx_rows, IDX_MINOR), jnp.int32),
            pltpu.VMEM((CHUNK, D), jnp.float32),
            pltpu.SemaphoreType.DMA,
        ],
    )
    def k(x_hbm, table_hbm, out_hbm, idx_v, rows_v, sem):
        wid = lax.axis_index("c") * NS + lax.axis_index("s")
        ch0 = wid * n_chunks
        pltpu.sync_copy(x_hbm.at[wid], idx_v)

        @pl.loop(0, n_chunks, unroll=1)
        def chunk_body(c):
            ch = ch0 + c
            span = ch // 2
            uc = ch % 2
            l = span // kb_per_l
            kb = span % kb_per_l
            descs = []
            for j in range(GATHERS_PER_CHUNK):
                descs.append(
                    pltpu.async_copy(
                        table_hbm.at[idx_v.at[c * GATHERS_PER_CHUNK + j]],
                        rows_v.at[pl.ds(j * IDX_MINOR, IDX_MINOR)],
                        sem,
                    )
                )
            for dsc in descs:
                dsc.wait()
            pltpu.sync_copy(rows_v.at[pl.ds(0, RB)], out_hbm.at[l, kb, :, 2 * uc])
            pltpu.sync_copy(
                rows_v.at[pl.ds(RB, RB)], out_hbm.at[l, kb, :, 2 * uc + 1]
            )

    return k(x_grouped, table)


def _tc_norm_t(z4, seg, gd, bb, L, B):
    nkb = B // KB

    def body(z_ref, seg_ref, g_ref, b_ref, o_ref):
        z = z_ref[...]
        s = seg_ref[...]
        gd32 = g_ref[...]
        bcol = b_ref[...][:, 0:1]
        s1 = jnp.dot(z, s, preferred_element_type=jnp.float32)
        s2 = jnp.dot(z * z, s, preferred_element_type=jnp.float32)
        mean = s1 * (1.0 / D)
        var = s2 * (1.0 / D) - mean * mean
        rstd = lax.rsqrt(var + EPS)
        u = (z - mean) * rstd
        for sseg in range(4):
            uq = u[:, sseg * D:(sseg + 1) * D]
            t = lax.dot_general(
                gd32, uq, (((1,), (1,)), ((), ())),
                preferred_element_type=jnp.float32,
            )
            o_ref[0, :, sseg * RB:(sseg + 1) * RB] = jnp.maximum(t + bcol, 0.0)

    return pl.pallas_call(
        body,
        grid=(L, nkb),
        in_specs=[
            pl.BlockSpec((RB, 128), lambda l, kb: (l * nkb + kb, 0)),
            pl.BlockSpec((128, 128), lambda l, kb: (0, 0)),
            pl.BlockSpec((D, D), lambda l, kb: (0, 0)),
            pl.BlockSpec((D, 128), lambda l, kb: (0, 0)),
        ],
        out_specs=pl.BlockSpec((1, D, KB), lambda l, kb: (l, 0, kb)),
        out_shape=jax.ShapeDtypeStruct((L, D, B), jnp.float32),
    )(z4, seg, gd, bb)


def kernel(x, table, gamma, beta):
    B, L = x.shape
    V = table.shape[0]
    N = B * L
    n_blocks = (V + TBLK - 1) // TBLK

    ident = jnp.asarray(np.eye(128), dtype=jnp.float32)
    t4 = _tc_pack_table(table.T, ident, n_blocks)
    tpack = t4.reshape(n_blocks * TBLK, D)

    xq = (
        jnp.left_shift(jnp.right_shift(x, 13), 13)
        | jnp.left_shift(x & 2047, 2)
        | (jnp.right_shift(x, 11) & 3)
    )
    xp = xq.T.reshape(NW, N // (NW * IDX_MINOR), IDX_MINOR)
    g = _sc_gather(xp, tpack, L, B)

    z4 = g.reshape(N // 4, 4 * D)
    seg = jnp.asarray(
        (np.arange(128)[:, None] // D) == (np.arange(128)[None, :] // D),
        dtype=jnp.float32,
    )
    gd = jnp.diag(gamma)
    bb = jnp.broadcast_to(beta[:, None], (D, 128))
    out_t = _tc_norm_t(z4, seg, gd, bb, L, B)
    return jnp.transpose(out_t, (2, 0, 1))
